# Initial kernel scaffold; baseline (speedup 1.0000x reference)
import jax
import jax.numpy as jnp
from jax.experimental import pallas as pl


def kernel(x, weights):
    raise NotImplementedError("write your pallas kernel here")


# speedup vs baseline: 1.9370x; 1.9370x over previous
import jax
import jax.numpy as jnp
from jax import lax
from jax.experimental import pallas as pl

_DIM_2 = 128
_BB = 256
_WIN = 5504
_BIG = 2**30


def _round_bf16(v):
    u = lax.bitcast_convert_type(v, jnp.uint32)
    r = (u + jnp.uint32(0x7FFF) + ((u >> 16) & jnp.uint32(1))) & jnp.uint32(0xFFFF0000)
    return lax.bitcast_convert_type(r, jnp.float32)


def _win_min_argmin(d2, lo, hi):
    blk = d2[:, lo:hi]
    m = jnp.min(blk, axis=1, keepdims=True)
    gidx = lax.broadcasted_iota(jnp.int32, blk.shape, 1) + lo
    bi = jnp.min(jnp.where(blk == m, gidx, _BIG), axis=1, keepdims=True)
    return m, bi


def _bmu_body(x_ref, w_ref, xsq_ref, wsq_ref, out_ref):
    mm = lax.dot_general(
        x_ref[...], w_ref[...],
        dimension_numbers=(((1,), (1,)), ((), ())),
        preferred_element_type=jnp.float32,
    )
    d2 = (xsq_ref[...] + wsq_ref[...]) - 2.0 * mm

    k = d2.shape[1]
    bounds = list(range(0, k, _WIN)) + [k]
    m0, i0 = _win_min_argmin(d2, bounds[0], bounds[1])
    cur_v = _round_bf16(jnp.sqrt(jnp.maximum(m0, 0.0)))
    cur_i = i0
    for w in range(1, len(bounds) - 1):
        mw, iw = _win_min_argmin(d2, bounds[w], bounds[w + 1])
        dw = jnp.sqrt(jnp.maximum(mw, 0.0))
        take = dw < cur_v
        cur_v = jnp.where(take, _round_bf16(dw), cur_v)
        cur_i = jnp.where(take, iw, cur_i)

    out_ref[...] = jnp.concatenate([cur_i >> 7, cur_i & (_DIM_2 - 1)], axis=1)


@jax.jit
def kernel(x, weights):
    b, d = x.shape
    k, _ = weights.shape
    nbb = b // _BB

    x_sq = jnp.sum(x * x, axis=1, keepdims=True)
    w_sq = jnp.sum(weights * weights, axis=1)[None, :]

    return pl.pallas_call(
        _bmu_body,
        grid=(nbb,),
        in_specs=[
            pl.BlockSpec((_BB, d), lambda ib: (ib, 0)),
            pl.BlockSpec((k, d), lambda ib: (0, 0)),
            pl.BlockSpec((_BB, 1), lambda ib: (ib, 0)),
            pl.BlockSpec((1, k), lambda ib: (0, 0)),
        ],
        out_specs=pl.BlockSpec((_BB, 2), lambda ib: (ib, 0)),
        out_shape=jax.ShapeDtypeStruct((b, 2), jnp.int32),
    )(x, weights, x_sq, w_sq)
